# baseline (speedup 1.0000x reference)
import functools

import jax
import jax.numpy as jnp
from jax import lax
from jax.experimental import pallas as pl
from jax.experimental.pallas import tpu as pltpu
from jax.experimental.pallas import tpu_sc as plsc

NC, NS, L = 2, 16, 16
NW = NC * NS

BATCH = 4096
SEQ = 200
EMBED_DIM = 64
TOTAL = BATCH * SEQ
PER_W = TOTAL // NW
CHUNK = 128
NCHUNK = PER_W // CHUNK


def _pos_encoding(seq_len, d):
    position = jnp.arange(0, seq_len, dtype=jnp.float32)[:, None]
    div_term = jnp.exp(jnp.arange(0, d, 2, dtype=jnp.float32) * -(jnp.log(10000.0) / d))
    enc = jnp.zeros((seq_len, d), dtype=jnp.float32)
    enc = enc.at[:, 0::2].set(jnp.sin(position * div_term))
    enc = enc.at[:, 1::2].set(jnp.cos(position * div_term[: d // 2]))
    return enc


def _body(weight_hbm, idx_hbm, enc_hbm, out_hbm, idx_v, rows_v, enc_v, sem):
    wid = lax.axis_index("s") * NC + lax.axis_index("c")
    base = wid * PER_W
    pltpu.sync_copy(enc_hbm, enc_v)

    def chunk_body(c, carry):
        gbase = base + c * CHUNK
        pltpu.sync_copy(idx_hbm.at[pl.ds(gbase, CHUNK)], idx_v)
        pltpu.async_copy(weight_hbm.at[idx_v], rows_v, sem).wait()
        off = (c * CHUNK) % SEQ

        def row_body(r, carry2):
            e = off + r
            for j in range(EMBED_DIM // L):
                sl = pl.ds(j * L, L)
                rows_v[r, sl] = rows_v[r, sl] + enc_v[e, sl]
            return carry2

        lax.fori_loop(0, CHUNK, row_body, 0, unroll=2)
        pltpu.sync_copy(rows_v, out_hbm.at[pl.ds(gbase, CHUNK)])
        return carry

    lax.fori_loop(0, NCHUNK, chunk_body, 0)


@jax.jit
def _embed(text, weight, enc2x):
    idx_flat = text.reshape(TOTAL).astype(jnp.int32)
    mesh = plsc.VectorSubcoreMesh(
        core_axis_name="c", subcore_axis_name="s", num_cores=NC, num_subcores=NS
    )
    out = pl.kernel(
        _body,
        out_type=jax.ShapeDtypeStruct((TOTAL, EMBED_DIM), jnp.float32),
        mesh=mesh,
        scratch_types=[
            pltpu.VMEM((CHUNK,), jnp.int32),
            pltpu.VMEM((CHUNK, EMBED_DIM), jnp.float32),
            pltpu.VMEM((2 * SEQ, EMBED_DIM), jnp.float32),
            pltpu.SemaphoreType.DMA,
        ],
        compiler_params=pltpu.CompilerParams(use_tc_tiling_on_sc=False),
    )(weight, idx_flat, enc2x)
    return out.reshape(BATCH, SEQ, EMBED_DIM)


def kernel(text, weight):
    enc = _pos_encoding(SEQ, EMBED_DIM)
    enc2x = jnp.concatenate([enc, enc], axis=0)
    return _embed(text, weight, enc2x)


# speedup vs baseline: 1.2344x; 1.2344x over previous
import functools

import jax
import jax.numpy as jnp
from jax import lax
from jax.experimental import pallas as pl
from jax.experimental.pallas import tpu as pltpu
from jax.experimental.pallas import tpu_sc as plsc

NC, NS, L = 2, 16, 16
NW = NC * NS

BATCH = 4096
SEQ = 200
EMBED_DIM = 64
TOTAL = BATCH * SEQ
PER_W = TOTAL // NW
CHUNK = 128
NCHUNK = PER_W // CHUNK
NBUF = 4
GROUPS = NCHUNK // NBUF


def _pos_encoding(seq_len, d):
    position = jnp.arange(0, seq_len, dtype=jnp.float32)[:, None]
    div_term = jnp.exp(jnp.arange(0, d, 2, dtype=jnp.float32) * -(jnp.log(10000.0) / d))
    enc = jnp.zeros((seq_len, d), dtype=jnp.float32)
    enc = enc.at[:, 0::2].set(jnp.sin(position * div_term))
    enc = enc.at[:, 1::2].set(jnp.cos(position * div_term[: d // 2]))
    return enc


def _body(weight_hbm, idx_hbm, enc_hbm, out_hbm, idx_all, enc_v,
          rows0, rows1, rows2, rows3, g0, g1, g2, g3, s0, s1, s2, s3):
    rows = [rows0, rows1, rows2, rows3]
    gsem = [g0, g1, g2, g3]
    ssem = [s0, s1, s2, s3]
    wid = lax.axis_index("s") * NC + lax.axis_index("c")
    base = wid * PER_W

    pltpu.sync_copy(idx_hbm.at[pl.ds(wid * NCHUNK, NCHUNK), :], idx_all)
    pltpu.sync_copy(enc_hbm, enc_v)

    def start_gather(c, b):
        pltpu.async_copy(weight_hbm.at[idx_all.at[c]], rows[b], gsem[b])

    def wait_gather(c, b):
        pltpu.make_async_copy(weight_hbm.at[idx_all.at[c]], rows[b], gsem[b]).wait()

    def start_store(c, b):
        pltpu.async_copy(rows[b], out_hbm.at[pl.ds(base + c * CHUNK, CHUNK)], ssem[b])

    def wait_store(c, b):
        pltpu.make_async_copy(
            rows[b], out_hbm.at[pl.ds(base + c * CHUNK, CHUNK)], ssem[b]
        ).wait()

    for b in range(NBUF - 1):
        start_gather(b, b)

    def group_body(g, carry):
        for b in range(NBUF):
            c = g * NBUF + b
            wait_gather(c, b)
            off = (c * CHUNK) % SEQ
            rv = rows[b]

            def row_body(r, carry2):
                e = off + r
                for j in range(EMBED_DIM // L):
                    sl = pl.ds(j * L, L)
                    rv[r, sl] = rv[r, sl] + enc_v[e, sl]
                return carry2

            lax.fori_loop(0, CHUNK, row_body, 0, unroll=8)
            start_store(c, b)
            pb = (b - 1) % NBUF
            if b == 0:

                @pl.when(g > 0)
                def _():
                    wait_store(g * NBUF - 1, pb)

                start_gather(c + NBUF - 1, pb)
            else:

                @pl.when(g < GROUPS - 1)
                def _():
                    wait_store(c - 1, pb)
                    start_gather(c + NBUF - 1, pb)

                @pl.when(g == GROUPS - 1)
                def _():
                    wait_store(c - 1, pb)

        return carry

    lax.fori_loop(0, GROUPS, group_body, 0)
    wait_store(NCHUNK - 1, (NCHUNK - 1) % NBUF)


@jax.jit
def _embed(text, weight, enc2x):
    idx2d = text.reshape(TOTAL // CHUNK, CHUNK).astype(jnp.int32)
    mesh = plsc.VectorSubcoreMesh(
        core_axis_name="c", subcore_axis_name="s", num_cores=NC, num_subcores=NS
    )
    out = pl.kernel(
        _body,
        out_type=jax.ShapeDtypeStruct((TOTAL, EMBED_DIM), jnp.float32),
        mesh=mesh,
        scratch_types=[
            pltpu.VMEM((NCHUNK, CHUNK), jnp.int32),
            pltpu.VMEM((2 * SEQ, EMBED_DIM), jnp.float32),
        ]
        + [pltpu.VMEM((CHUNK, EMBED_DIM), jnp.float32) for _ in range(NBUF)]
        + [pltpu.SemaphoreType.DMA for _ in range(2 * NBUF)],
        compiler_params=pltpu.CompilerParams(use_tc_tiling_on_sc=False),
    )(weight, idx2d, enc2x)
    return out.reshape(BATCH, SEQ, EMBED_DIM)


def kernel(text, weight):
    enc = _pos_encoding(SEQ, EMBED_DIM)
    enc2x = jnp.concatenate([enc, enc], axis=0)
    return _embed(text, weight, enc2x)


# speedup vs baseline: 1.5534x; 1.2584x over previous
import functools

import jax
import jax.numpy as jnp
from jax import lax
from jax.experimental import pallas as pl
from jax.experimental.pallas import tpu as pltpu
from jax.experimental.pallas import tpu_sc as plsc

NC, NS, L = 2, 16, 16
NW = NC * NS

BATCH = 4096
SEQ = 200
EMBED_DIM = 64
TOTAL = BATCH * SEQ
PER_W = TOTAL // NW
CHUNK = 128
NCHUNK = PER_W // CHUNK
NBUF = 8
GROUPS = NCHUNK // NBUF
ENC_ROWS = SEQ + CHUNK - 8


def _pos_encoding(seq_len, d):
    position = jnp.arange(0, seq_len, dtype=jnp.float32)[:, None]
    div_term = jnp.exp(jnp.arange(0, d, 2, dtype=jnp.float32) * -(jnp.log(10000.0) / d))
    enc = jnp.zeros((seq_len, d), dtype=jnp.float32)
    enc = enc.at[:, 0::2].set(jnp.sin(position * div_term))
    enc = enc.at[:, 1::2].set(jnp.cos(position * div_term[: d // 2]))
    return enc


def _body(weight_hbm, idx_hbm, enc_hbm, out_hbm, idx_all, enc_v,
          rows_bufs, gsems, ssems):
    wid = lax.axis_index("s") * NC + lax.axis_index("c")
    base = wid * PER_W

    pltpu.sync_copy(idx_hbm.at[pl.ds(wid * NCHUNK, NCHUNK), :], idx_all)
    pltpu.sync_copy(enc_hbm, enc_v)

    def start_gather(c, b):
        pltpu.async_copy(weight_hbm.at[idx_all.at[c]], rows_bufs[b], gsems[b])

    def wait_gather(c, b):
        pltpu.make_async_copy(
            weight_hbm.at[idx_all.at[c]], rows_bufs[b], gsems[b]
        ).wait()

    def start_store(c, b):
        pltpu.async_copy(
            rows_bufs[b], out_hbm.at[pl.ds(base + c * CHUNK, CHUNK)], ssems[b]
        )

    def wait_store(c, b):
        pltpu.make_async_copy(
            rows_bufs[b], out_hbm.at[pl.ds(base + c * CHUNK, CHUNK)], ssems[b]
        ).wait()

    for b in range(NBUF - 2):
        start_gather(b, b)

    def group_body(g, carry):
        for b in range(NBUF):
            c = g * NBUF + b
            wait_gather(c, b)
            off = (c * CHUNK) % SEQ
            rv = rows_bufs[b]

            @plsc.parallel_loop(0, CHUNK, unroll=8)
            def _(r):
                e = off + r
                for j in range(EMBED_DIM // L):
                    sl = pl.ds(j * L, L)
                    rv[r, sl] = rv[r, sl] + enc_v[e, sl]

            start_store(c, b)
            jb = (b - 2) % NBUF
            jc = c + NBUF - 2
            if b < 2:

                @pl.when(g > 0)
                def _():
                    wait_store(c - 2, jb)

                start_gather(jc, jb)
            else:
                wait_store(c - 2, jb)

                @pl.when(g < GROUPS - 1)
                def _():
                    start_gather(jc, jb)

        return carry

    lax.fori_loop(0, GROUPS, group_body, 0)
    wait_store(NCHUNK - 2, (NCHUNK - 2) % NBUF)
    wait_store(NCHUNK - 1, (NCHUNK - 1) % NBUF)


@jax.jit
def _embed(text, weight, enc_ext):
    idx2d = text.reshape(TOTAL // CHUNK, CHUNK).astype(jnp.int32)
    mesh = plsc.VectorSubcoreMesh(
        core_axis_name="c", subcore_axis_name="s", num_cores=NC, num_subcores=NS
    )

    def body(weight_hbm, idx_hbm, enc_hbm, out_hbm, idx_all, enc_v, *rest):
        rows_bufs = rest[:NBUF]
        gsems = rest[NBUF:2 * NBUF]
        ssems = rest[2 * NBUF:]
        _body(weight_hbm, idx_hbm, enc_hbm, out_hbm, idx_all, enc_v,
              rows_bufs, gsems, ssems)

    out = pl.kernel(
        body,
        out_type=jax.ShapeDtypeStruct((TOTAL, EMBED_DIM), jnp.float32),
        mesh=mesh,
        scratch_types=[
            pltpu.VMEM((NCHUNK, CHUNK), jnp.int32),
            pltpu.VMEM((ENC_ROWS, EMBED_DIM), jnp.float32),
        ]
        + [pltpu.VMEM((CHUNK, EMBED_DIM), jnp.float32) for _ in range(NBUF)]
        + [pltpu.SemaphoreType.DMA for _ in range(2 * NBUF)],
        compiler_params=pltpu.CompilerParams(use_tc_tiling_on_sc=False),
    )(weight, idx2d, enc_ext)
    return out.reshape(BATCH, SEQ, EMBED_DIM)


def kernel(text, weight):
    enc = _pos_encoding(SEQ, EMBED_DIM)
    enc_ext = jnp.concatenate([enc, enc[: ENC_ROWS - SEQ]], axis=0)
    return _embed(text, weight, enc_ext)
